# baseline (speedup 1.0000x reference)
import jax
import jax.numpy as jnp
from jax.experimental import pallas as pl
from jax.experimental.pallas import tpu as pltpu

B, N, F, C = 8, 512, 128, 16
NG = 10
FCN = 64
_WMAP = [0, 1, 2, 3, 4, 5, 6, 7, 6, 7]


def _body(*refs):
    x_refs = refs[0:NG]
    a_refs = refs[NG:2 * NG]
    w_refs = refs[2 * NG:2 * NG + 8]
    wdt_hbm = refs[2 * NG + 8]
    bd0_ref = refs[2 * NG + 9]
    wd1t_ref = refs[2 * NG + 10]
    bd1_ref = refs[2 * NG + 11]
    out_ref = refs[2 * NG + 12]
    xct_sc, wdt_sc, sem = refs[2 * NG + 13:]

    b = pl.program_id(0)

    @pl.when(b == 0)
    def _():
        pltpu.make_async_copy(wdt_hbm, wdt_sc, sem).start()

    for i in range(NG):
        xw = jax.lax.dot_general(
            x_refs[i][0], w_refs[_WMAP[i]][...],
            (((1,), (1,)), ((), ())),
            preferred_element_type=jnp.float32)
        h = jnp.maximum(jnp.dot(a_refs[i][0], xw,
                                preferred_element_type=jnp.float32), 0.0)
        xct_sc[b, i] = h.T

    @pl.when(b == B - 1)
    def _():
        pltpu.make_async_copy(wdt_hbm, wdt_sc, sem).wait()
        acc = None
        for i in range(NG):
            lhs3 = xct_sc[:, i]
            t = jnp.transpose(lhs3, (2, 1, 0))
            f = t.reshape(N * C, B)
            p = jax.lax.dot_general(
                wdt_sc[:, i * N * C:(i + 1) * N * C], f,
                (((1,), (0,)), ((), ())),
                preferred_element_type=jnp.float32)
            acc = p if acc is None else acc + p
        o1t = jnp.maximum(acc + bd0_ref[...], 0.0)
        out_ref[...] = jax.lax.dot_general(
            wd1t_ref[...], o1t, (((1,), (0,)), ((), ())),
            preferred_element_type=jnp.float32) + bd1_ref[...]


def kernel(x1, a1, x2, a2, x3, a3, x4, a4, x5, a5, x6, a6, x7, a7, x8, a8,
           x9, a9, x10, a10, Wg0, Wg1, Wg2, Wg3, Wg4, Wg5, Wg6, Wg7,
           Wd0, bd0, Wd1, bd1):
    xs = [x1, x2, x3, x4, x5, x6, x7, x8, x9, x10]
    adjs = [a1, a2, a3, a4, a5, a6, a7, a8, a9, a10]
    wgts = [W.T for W in (Wg0, Wg1, Wg2, Wg3, Wg4, Wg5, Wg6, Wg7)]

    o2t = pl.pallas_call(
        _body,
        grid=(B,),
        in_specs=(
            [pl.BlockSpec((1, N, F), lambda b: (b, 0, 0)) for _ in range(NG)]
            + [pl.BlockSpec((1, N, N), lambda b: (b, 0, 0)) for _ in range(NG)]
            + [pl.BlockSpec((C, F), lambda b: (0, 0)) for _ in range(8)]
            + [
                pl.BlockSpec(memory_space=pltpu.MemorySpace.HBM),
                pl.BlockSpec((FCN, 1), lambda b: (0, 0)),
                pl.BlockSpec((1, FCN), lambda b: (0, 0)),
                pl.BlockSpec((1, 1), lambda b: (0, 0)),
            ]
        ),
        out_specs=pl.BlockSpec((1, B), lambda b: (0, 0)),
        out_shape=jax.ShapeDtypeStruct((1, B), jnp.float32),
        scratch_shapes=[
            pltpu.VMEM((B, NG, C, N), jnp.float32),
            pltpu.VMEM((FCN, NG * N * C), jnp.float32),
            pltpu.SemaphoreType.DMA,
        ],
    )(*xs, *adjs, *wgts, Wd0.T, bd0.reshape(FCN, 1), Wd1.T, bd1.reshape(1, 1))
    return o2t.T


# speedup vs baseline: 1.0608x; 1.0608x over previous
import jax
import jax.numpy as jnp
from jax.experimental import pallas as pl
from jax.experimental.pallas import tpu as pltpu

B, N, F, C = 8, 512, 128, 16
NG = 10
FCN = 64
_WMAP = [0, 1, 2, 3, 4, 5, 6, 7, 6, 7]


def _gcn_body(*refs):
    x_refs = refs[0:NG]
    a_refs = refs[NG:2 * NG]
    w_refs = refs[2 * NG:2 * NG + 8]
    out_ref = refs[-1]
    for i in range(NG):
        xw = jax.lax.dot_general(
            x_refs[i][0], w_refs[_WMAP[i]][...],
            (((1,), (1,)), ((), ())),
            preferred_element_type=jnp.float32)
        h = jnp.maximum(jnp.dot(a_refs[i][0], xw,
                                preferred_element_type=jnp.float32), 0.0)
        out_ref[0, i] = h.T


def _head_body(xct_ref, wdt_ref, bd0_ref, wd1t_ref, bd1_ref, out_ref, acc_ref):
    i = pl.program_id(0)
    lhs3 = xct_ref[:, 0].astype(jnp.bfloat16)
    t = jnp.transpose(lhs3, (2, 1, 0))
    f = t.reshape(N * C, B)
    p = jax.lax.dot_general(wdt_ref[...].astype(jnp.bfloat16), f,
                            (((1,), (0,)), ((), ())),
                            preferred_element_type=jnp.float32)

    @pl.when(i == 0)
    def _():
        acc_ref[...] = p

    @pl.when(i > 0)
    def _():
        acc_ref[...] = acc_ref[...] + p

    @pl.when(i == NG - 1)
    def _():
        o1t = jnp.maximum(acc_ref[...] + bd0_ref[...], 0.0)
        out_ref[...] = jax.lax.dot_general(
            wd1t_ref[...], o1t, (((1,), (0,)), ((), ())),
            preferred_element_type=jnp.float32) + bd1_ref[...]


def kernel(x1, a1, x2, a2, x3, a3, x4, a4, x5, a5, x6, a6, x7, a7, x8, a8,
           x9, a9, x10, a10, Wg0, Wg1, Wg2, Wg3, Wg4, Wg5, Wg6, Wg7,
           Wd0, bd0, Wd1, bd1):
    xs = [x1, x2, x3, x4, x5, x6, x7, x8, x9, x10]
    adjs = [a1, a2, a3, a4, a5, a6, a7, a8, a9, a10]
    wgts = [W.T for W in (Wg0, Wg1, Wg2, Wg3, Wg4, Wg5, Wg6, Wg7)]

    xct = pl.pallas_call(
        _gcn_body,
        grid=(B,),
        in_specs=(
            [pl.BlockSpec((1, N, F), lambda b: (b, 0, 0)) for _ in range(NG)]
            + [pl.BlockSpec((1, N, N), lambda b: (b, 0, 0)) for _ in range(NG)]
            + [pl.BlockSpec((C, F), lambda b: (0, 0)) for _ in range(8)]
        ),
        out_specs=pl.BlockSpec((1, NG, C, N), lambda b: (b, 0, 0, 0)),
        out_shape=jax.ShapeDtypeStruct((B, NG, C, N), jnp.float32),
    )(*xs, *adjs, *wgts)

    o2t = pl.pallas_call(
        _head_body,
        grid=(NG,),
        in_specs=(
            pl.BlockSpec((B, 1, C, N), lambda i: (0, i, 0, 0)),
            pl.BlockSpec((FCN, N * C), lambda i: (0, i)),
            pl.BlockSpec((FCN, 1), lambda i: (0, 0)),
            pl.BlockSpec((1, FCN), lambda i: (0, 0)),
            pl.BlockSpec((1, 1), lambda i: (0, 0)),
        ),
        out_specs=pl.BlockSpec((1, B), lambda i: (0, 0)),
        out_shape=jax.ShapeDtypeStruct((1, B), jnp.float32),
        scratch_shapes=[pltpu.VMEM((FCN, B), jnp.float32)],
    )(xct, Wd0.T, bd0.reshape(FCN, 1), Wd1.T, bd1.reshape(1, 1))
    return o2t.T


# speedup vs baseline: 1.0732x; 1.0118x over previous
import jax
import jax.numpy as jnp
from jax.experimental import pallas as pl
from jax.experimental.pallas import tpu as pltpu

B, N, F, C = 8, 512, 128, 16
NG = 10
FCN = 64
_WMAP = [0, 1, 2, 3, 4, 5, 6, 7, 6, 7]


def _gcn_body(*refs):
    x_refs = refs[0:NG]
    a_refs = refs[NG:2 * NG]
    w_refs = refs[2 * NG:2 * NG + 8]
    out_ref = refs[-1]
    for i in range(NG):
        xw = jax.lax.dot_general(
            x_refs[i][0], w_refs[_WMAP[i]][...],
            (((1,), (1,)), ((), ())),
            preferred_element_type=jnp.float32)
        h = jnp.maximum(jnp.dot(a_refs[i][0], xw,
                                preferred_element_type=jnp.float32), 0.0)
        out_ref[0, i] = h.astype(jnp.bfloat16).T


def _head_body(xct_ref, wdt_ref, bd0_ref, wd1t_ref, bd1_ref, out_ref, acc_ref):
    i = pl.program_id(0)
    lhs3 = xct_ref[:, 0]
    t = jnp.transpose(lhs3, (2, 1, 0))
    f = t.reshape(N * C, B)
    p = jax.lax.dot_general(wdt_ref[...].astype(jnp.bfloat16), f,
                            (((1,), (0,)), ((), ())),
                            preferred_element_type=jnp.float32)

    @pl.when(i == 0)
    def _():
        acc_ref[...] = p

    @pl.when(i > 0)
    def _():
        acc_ref[...] = acc_ref[...] + p

    @pl.when(i == NG - 1)
    def _():
        o1t = jnp.maximum(acc_ref[...] + bd0_ref[...], 0.0)
        out_ref[...] = jax.lax.dot_general(
            wd1t_ref[...], o1t, (((1,), (0,)), ((), ())),
            preferred_element_type=jnp.float32) + bd1_ref[...]


def kernel(x1, a1, x2, a2, x3, a3, x4, a4, x5, a5, x6, a6, x7, a7, x8, a8,
           x9, a9, x10, a10, Wg0, Wg1, Wg2, Wg3, Wg4, Wg5, Wg6, Wg7,
           Wd0, bd0, Wd1, bd1):
    xs = [x1, x2, x3, x4, x5, x6, x7, x8, x9, x10]
    adjs = [a1, a2, a3, a4, a5, a6, a7, a8, a9, a10]
    wgts = [W.T for W in (Wg0, Wg1, Wg2, Wg3, Wg4, Wg5, Wg6, Wg7)]

    xct = pl.pallas_call(
        _gcn_body,
        grid=(B,),
        in_specs=(
            [pl.BlockSpec((1, N, F), lambda b: (b, 0, 0)) for _ in range(NG)]
            + [pl.BlockSpec((1, N, N), lambda b: (b, 0, 0)) for _ in range(NG)]
            + [pl.BlockSpec((C, F), lambda b: (0, 0)) for _ in range(8)]
        ),
        out_specs=pl.BlockSpec((1, NG, C, N), lambda b: (b, 0, 0, 0)),
        out_shape=jax.ShapeDtypeStruct((B, NG, C, N), jnp.bfloat16),
    )(*xs, *adjs, *wgts)

    o2t = pl.pallas_call(
        _head_body,
        grid=(NG,),
        in_specs=(
            pl.BlockSpec((B, 1, C, N), lambda i: (0, i, 0, 0)),
            pl.BlockSpec((FCN, N * C), lambda i: (0, i)),
            pl.BlockSpec((FCN, 1), lambda i: (0, 0)),
            pl.BlockSpec((1, FCN), lambda i: (0, 0)),
            pl.BlockSpec((1, 1), lambda i: (0, 0)),
        ),
        out_specs=pl.BlockSpec((1, B), lambda i: (0, 0)),
        out_shape=jax.ShapeDtypeStruct((1, B), jnp.float32),
        scratch_shapes=[pltpu.VMEM((FCN, B), jnp.float32)],
    )(xct, Wd0.T, bd0.reshape(FCN, 1), Wd1.T, bd1.reshape(1, 1))
    return o2t.T
